# baseline (speedup 1.0000x reference)
import functools

import jax
import jax.numpy as jnp
from jax import lax
from jax.experimental import pallas as pl
from jax.experimental.pallas import tpu as pltpu
from jax.experimental.pallas import tpu_sc as plsc

E = 8
K = 2
H = 1024
MT = 1024
T = 512
NW = 32


def _router_body(x_ref, wr_ref, br_ref, w_ref, e_ref):
    scores = jax.lax.dot_general(
        x_ref[...], wr_ref[...], (((1,), (1,)), ((), ())),
        preferred_element_type=jnp.float32,
    ) + br_ref[...]
    m = jnp.max(scores, axis=1, keepdims=True)
    p = jnp.exp(scores - m)
    p = p / jnp.sum(p, axis=1, keepdims=True)
    iota = jax.lax.broadcasted_iota(jnp.int32, p.shape, 1)
    m0 = jnp.max(p, axis=1, keepdims=True)
    a0 = jnp.min(jnp.where(p == m0, iota, E), axis=1, keepdims=True)
    p1m = jnp.where(iota == a0, -1.0, p)
    m1 = jnp.max(p1m, axis=1, keepdims=True)
    a1 = jnp.min(jnp.where(p1m == m1, iota, E), axis=1, keepdims=True)
    ws = m0 + m1
    w_ref[...] = jnp.concatenate([m0 / ws, m1 / ws], axis=1)
    e_ref[...] = jnp.concatenate([a0, a1], axis=1)


def _router(xf, Wr, br2):
    M = xf.shape[0]
    return pl.pallas_call(
        _router_body,
        grid=(M // MT,),
        in_specs=[
            pl.BlockSpec((MT, H), lambda t: (t, 0)),
            pl.BlockSpec((E, H), lambda t: (0, 0)),
            pl.BlockSpec((1, E), lambda t: (0, 0)),
        ],
        out_specs=[
            pl.BlockSpec((MT, K), lambda t: (t, 0)),
            pl.BlockSpec((MT, K), lambda t: (t, 0)),
        ],
        out_shape=[
            jax.ShapeDtypeStruct((M, K), jnp.float32),
            jax.ShapeDtypeStruct((M, K), jnp.int32),
        ],
    )(xf, Wr, br2)


def _make_gather(M, Apad):
    rows_w = Apad // NW
    CH = 64
    nch = rows_w // CH
    mesh = plsc.VectorSubcoreMesh(core_axis_name="c", subcore_axis_name="s")

    @functools.partial(
        pl.kernel,
        out_type=jax.ShapeDtypeStruct((Apad, H), jnp.float32),
        mesh=mesh,
        scratch_types=[
            pltpu.VMEM((CH,), jnp.int32),
            pltpu.VMEM((CH, H), jnp.float32),
            pltpu.SemaphoreType.DMA,
        ],
    )
    def gather_k(x_hbm, idx_hbm, out_hbm, idx_v, rows_v, sem):
        wid = lax.axis_index("s") * 2 + lax.axis_index("c")
        base = wid * rows_w
        for c in range(nch):
            off = base + c * CH
            pltpu.sync_copy(idx_hbm.at[pl.ds(off, CH)], idx_v)
            pltpu.async_copy(x_hbm.at[idx_v], rows_v, sem).wait()
            pltpu.sync_copy(rows_v, out_hbm.at[pl.ds(off, CH)])

    return gather_k


def _grouped_body(ue_ref, um_ref, xs_ref, we_ref, be_ref, w_ref, ys_ref):
    del um_ref
    xb = xs_ref[...].astype(jnp.bfloat16)
    wb = we_ref[0].astype(jnp.bfloat16)
    y = jax.lax.dot_general(
        xb, wb, (((1,), (1,)), ((), ())),
        preferred_element_type=jnp.float32,
    ) + be_ref[0, 0]
    ys_ref[...] = y * w_ref[...]


def _grouped(xs, We, be3, w_pad2, u_e, u_m, G):
    Apad = xs.shape[0]
    return pl.pallas_call(
        _grouped_body,
        grid_spec=pltpu.PrefetchScalarGridSpec(
            num_scalar_prefetch=2,
            grid=(G,),
            in_specs=[
                pl.BlockSpec((T, H), lambda g, ue, um: (um[g], 0)),
                pl.BlockSpec((1, H, H), lambda g, ue, um: (ue[g], 0, 0)),
                pl.BlockSpec((1, 1, H), lambda g, ue, um: (ue[g], 0, 0)),
                pl.BlockSpec((T, 1), lambda g, ue, um: (um[g], 0)),
            ],
            out_specs=pl.BlockSpec((T, H), lambda g, ue, um: (um[g], 0)),
        ),
        out_shape=jax.ShapeDtypeStruct((Apad, H), jnp.float32),
    )(u_e, u_m, xs, We, be3, w_pad2)


def _make_combine(M, Apad):
    tok_w = M // NW
    CH = 32
    nch = tok_w // CH
    mesh = plsc.VectorSubcoreMesh(core_axis_name="c", subcore_axis_name="s")

    @functools.partial(
        pl.kernel,
        out_type=jax.ShapeDtypeStruct((M, H), jnp.float32),
        mesh=mesh,
        scratch_types=[
            pltpu.VMEM((CH,), jnp.int32),
            pltpu.VMEM((CH,), jnp.int32),
            pltpu.VMEM((CH, H), jnp.float32),
            pltpu.VMEM((CH, H), jnp.float32),
            pltpu.SemaphoreType.DMA,
        ],
    )
    def combine_k(ys_hbm, pa_hbm, pb_hbm, out_hbm, ia_v, ib_v, ba_v, bb_v,
                  sem):
        wid = lax.axis_index("s") * 2 + lax.axis_index("c")
        base = wid * tok_w
        for c in range(nch):
            off = base + c * CH
            pltpu.sync_copy(pa_hbm.at[pl.ds(off, CH)], ia_v)
            pltpu.sync_copy(pb_hbm.at[pl.ds(off, CH)], ib_v)
            pltpu.async_copy(ys_hbm.at[ia_v], ba_v, sem).wait()
            pltpu.async_copy(ys_hbm.at[ib_v], bb_v, sem).wait()

            def row_add(r, _):
                for j in range(H // 16):
                    s = pl.ds(j * 16, 16)
                    ba_v[r, s] = ba_v[r, s] + bb_v[r, s]
                return 0

            lax.fori_loop(0, CH, row_add, 0)
            pltpu.sync_copy(ba_v, out_hbm.at[pl.ds(off, CH)])

    return combine_k


def kernel(x, Wr, br, We, be):
    B, S, Hx = x.shape
    M = B * S
    A = M * K
    G = A // T + E
    Apad = G * T
    xf = x.reshape(M, Hx)
    br2 = br.reshape(1, E)
    be3 = be.reshape(E, 1, Hx)

    w2, e2 = _router(xf, Wr, br2)
    ef = e2.reshape(A)
    wf = w2.reshape(A)

    oh = (ef[:, None] == jnp.arange(E)[None, :]).astype(jnp.int32)
    c = jnp.cumsum(oh, axis=0)
    counts = c[-1]
    rank = jnp.take_along_axis(c, ef[:, None], 1)[:, 0] - 1
    tpe = (counts + T - 1) // T
    poff = jnp.concatenate(
        [jnp.zeros((1,), dtype=tpe.dtype), jnp.cumsum(tpe)[:-1]]) * T
    pos = (poff[ef] + rank).astype(jnp.int32)
    used = tpe.sum()
    u_e = jnp.repeat(jnp.arange(E), tpe, total_repeat_length=G)
    u_e = jnp.where(jnp.arange(G) < used, u_e, u_e[used - 1]).astype(jnp.int32)
    u_m = jnp.minimum(jnp.arange(G), used - 1).astype(jnp.int32)
    tok_pad = jnp.zeros((Apad,), jnp.int32).at[pos].set(
        jnp.arange(A, dtype=jnp.int32) // K)
    w_pad = jnp.zeros((Apad,), jnp.float32).at[pos].set(wf)
    posA = pos[0::2]
    posB = pos[1::2]

    xs = _make_gather(M, Apad)(xf, tok_pad)
    ys = _grouped(xs, We, be3, w_pad.reshape(Apad, 1), u_e, u_m, G)
    out = _make_combine(M, Apad)(ys, posA, posB)
    return out.reshape(B, S, Hx)
